# baseline (speedup 1.0000x reference)
import functools

import jax
import jax.numpy as jnp
from jax import lax
from jax.experimental import pallas as pl
from jax.experimental.pallas import tpu as pltpu
from jax.experimental.pallas import tpu_sc as plsc

BATCH = 16384
DIM = 64
MARGIN = 1.0
PAIR = 128

NW = 32
NROWS = 1000000

CHUNK = 512
NCH = NROWS // CHUNK
TAILE = NROWS - NCH * CHUNK
CPW = 61
TSLOTS = 62

N_ENT = 4 * BATCH
N_REL = 2 * BATCH
N_ALL = N_ENT + N_REL
DUMP = N_ALL
OUT_ROWS = 100352

LBLK = 512


def _iota16():
    return lax.iota(jnp.int32, 16)


def _lane(vec, j):
    return jnp.sum(jnp.where(_iota16() == j, vec, 0))


def _sc_extract_body(entT, relT, sie, sde, sir, sdr, wse, wsr, out,
                     slab, tslab, lbi, lbd, wsv, staging, dst2d,
                     sem0, sem1, sc0, sc1, sc2, sc3):
    wid = lax.axis_index("s") * 2 + lax.axis_index("c")
    slab_sems = (sem0, sem1)
    sc_sems = (sc0, sc1, sc2, sc3)

    def ws_at(ws_ref, k):
        pltpu.sync_copy(ws_ref.at[pl.ds((k // 16) * 16, 16)], wsv)
        return _lane(wsv[...], k % 16)

    def scatter_wait(r_static, nscat):
        @pl.when(nscat >= 4)
        def _():
            pltpu.make_async_copy(
                staging.at[r_static],
                out.at[dst2d.at[r_static]],
                sc_sems[r_static]).wait()

    def scatter_start(r_static):
        pltpu.make_async_copy(
            staging.at[r_static],
            out.at[dst2d.at[r_static]],
            sc_sems[r_static]).start()

    def window_passes(slab_ref, width, sidx, sdst, s0, s1, e0, live0, state):
        hi = e0 + width

        def cond(st):
            return st[3]

        def body(st):
            wp, blk, nscat, _ = st
            wblk = (16 * wp) // LBLK

            @pl.when(wblk != blk)
            def _():
                pltpu.sync_copy(sidx.at[pl.ds(wblk * LBLK, LBLK)], lbi)
                pltpu.sync_copy(sdst.at[pl.ds(wblk * LBLK, LBLK)], lbd)

            ow = (16 * wp) - wblk * LBLK
            idxw = lbi[pl.ds(ow, 16)]
            dstw = lbd[pl.ds(ow, 16)]
            posv = _iota16() + 16 * wp
            valid = ((posv >= s0) & (posv < s1)
                     & (idxw >= e0) & (idxw < hi))
            cols = jnp.clip(idxw - e0, 0, width - 1)

            r = nscat % 4
            for rr in range(4):
                @pl.when(r == rr)
                def _(rr=rr):
                    scatter_wait(rr, nscat)
            for j in range(16):
                cj = jnp.full((16,), _lane(cols, j), jnp.int32)
                for g in range(4):
                    v = plsc.load_gather(
                        slab_ref, [_iota16() + 16 * g, cj])
                    staging[r, j, pl.ds(16 * g, 16)] = v
            dst2d[r, pl.ds(0, 16)] = jnp.where(valid, dstw, DUMP)
            for rr in range(4):
                @pl.when(r == rr)
                def _(rr=rr):
                    scatter_start(rr)

            bad = jnp.max(jnp.where((idxw >= hi) & (posv < s1), 1, 0))
            done = bad == 0
            wp2 = jnp.where(done, wp + 1, wp)
            more = done & (16 * wp2 < s1)
            return (wp2, wblk, nscat + 1, more)

        wp, blk, nscat, _ = lax.while_loop(
            cond, body, (state[0], state[1], state[2],
                         live0 & (16 * state[0] < s1)))
        return (wp, blk, nscat)

    def phase(tab, sidx, sdst, ws_ref, nscat_in):
        s0 = ws_at(ws_ref, wid)
        s1 = ws_at(ws_ref, wid + 1)
        c_lo = wid * CPW
        c_hi = jnp.minimum(c_lo + CPW + 1, NCH)

        def slab_copy(c, b):
            return pltpu.make_async_copy(
                tab.at[:, pl.ds(c * CHUNK, CHUNK)], slab.at[b], slab_sems[b])

        slab_copy(c_lo, 0).start()

        state0 = (s0 // 16, jnp.int32(-1), nscat_in)

        @pl.loop(0, TSLOTS // 2, init_carry=state0)
        def chunk_loop(tt, carry):
            for b in range(2):
                t = 2 * tt + b
                c = c_lo + t

                @pl.when(c + 1 < c_hi)
                def _():
                    slab_copy(c + 1, 1 - b).start()

                live = c < c_hi

                @pl.when(live)
                def _():
                    slab_copy(c, b).wait()

                carry = window_passes(
                    slab.at[b], CHUNK, sidx, sdst, s0, s1,
                    c * CHUNK, live, carry)
            return carry

        wp, blk, nscat = chunk_loop
        live_t = wid == NW - 1

        @pl.when(live_t)
        def _():
            pltpu.sync_copy(tab.at[:, pl.ds(NCH * CHUNK, TAILE)], tslab)

        wp, blk, nscat = window_passes(
            tslab, TAILE, sidx, sdst, s0, s1,
            NCH * CHUNK, live_t, (wp, blk, nscat))
        return nscat

    nscat = phase(entT, sie, sde, wse, jnp.int32(0))
    nscat = phase(relT, sir, sdr, wsr, nscat)

    for d in range(1, 5):
        k = nscat - d
        r = k % 4
        for rr in range(4):
            @pl.when((k >= 0) & (r == rr))
            def _(rr=rr):
                pltpu.make_async_copy(
                    staging.at[rr], out.at[dst2d.at[rr]],
                    sc_sems[rr]).wait()


def _make_sc_extract():
    mesh = plsc.VectorSubcoreMesh(core_axis_name="c", subcore_axis_name="s")
    return functools.partial(
        pl.kernel, mesh=mesh,
        compiler_params=pltpu.CompilerParams(needs_layout_passes=False),
        out_type=jax.ShapeDtypeStruct((OUT_ROWS, PAIR), jnp.float32),
        scratch_types=[
            pltpu.VMEM((2, DIM, CHUNK), jnp.float32),
            pltpu.VMEM((DIM, TAILE), jnp.float32),
            pltpu.VMEM((LBLK,), jnp.int32),
            pltpu.VMEM((LBLK,), jnp.int32),
            pltpu.VMEM((16,), jnp.int32),
            pltpu.VMEM((4, 16, PAIR), jnp.float32),
            pltpu.VMEM((4, 16), jnp.int32),
            pltpu.SemaphoreType.DMA,
            pltpu.SemaphoreType.DMA,
            pltpu.SemaphoreType.DMA,
            pltpu.SemaphoreType.DMA,
            pltpu.SemaphoreType.DMA,
            pltpu.SemaphoreType.DMA,
        ],
    )(_sc_extract_body)


_sc_extract = _make_sc_extract()

CB = 2048
NBLK = BATCH // CB


def _score_body(ph, pt, pr, nh, nt, nr, out):
    k = pl.program_id(0)

    def score(h_ref, t_ref, r_ref):
        h = h_ref[...][:, :DIM]
        t = t_ref[...][:, :DIM]
        r = r_ref[...][:, :DIM]
        hn = h / jnp.sqrt(jnp.sum(h * h, axis=1, keepdims=True))
        tn = t / jnp.sqrt(jnp.sum(t * t, axis=1, keepdims=True))
        diff = hn + r - tn
        return jnp.sqrt(jnp.sum(diff * diff, axis=1))

    ps = score(ph, pt, pr)
    ns = score(nh, nt, nr)
    part = jnp.sum(jnp.maximum(MARGIN + ps - ns, 0.0)).reshape(1, 1)
    prev = jnp.where(k == 0, jnp.zeros((1, 1), jnp.float32), out[...])
    total = prev + part
    out[...] = jnp.where(k == NBLK - 1, total / BATCH, total)


def _tc_score(rows):
    def blk(off):
        return pl.BlockSpec((CB, PAIR), lambda k, o=off: (k + o, 0))

    out = pl.pallas_call(
        _score_body,
        grid=(NBLK,),
        in_specs=[blk(0), blk(NBLK), blk(4 * NBLK), blk(2 * NBLK),
                  blk(3 * NBLK), blk(5 * NBLK)],
        out_specs=pl.BlockSpec((1, 1), lambda k: (0, 0)),
        out_shape=jax.ShapeDtypeStruct((1, 1), jnp.float32),
    )(rows, rows, rows, rows, rows, rows)
    return out.reshape(())


def kernel(pos_x, neg_x, ent_table, rel_table):
    ent_idx = jnp.concatenate(
        [pos_x[:, 0], pos_x[:, 1], neg_x[:, 0], neg_x[:, 1]])
    rel_idx = jnp.concatenate([pos_x[:, 2], neg_x[:, 2]])

    eperm = jnp.argsort(ent_idx)
    sie = ent_idx[eperm]
    sde = eperm.astype(jnp.int32)
    rperm = jnp.argsort(rel_idx)
    sir = rel_idx[rperm]
    sdr = rperm.astype(jnp.int32) + N_ENT

    bounds = jnp.concatenate(
        [jnp.arange(32, dtype=jnp.int32) * (CPW * CHUNK),
         jnp.array([NROWS], jnp.int32)])
    wse = jnp.zeros((48,), jnp.int32).at[:33].set(
        jnp.searchsorted(sie, bounds).astype(jnp.int32))
    wsr = jnp.zeros((48,), jnp.int32).at[:33].set(
        jnp.searchsorted(sir, bounds).astype(jnp.int32))

    rows = _sc_extract(ent_table.T, rel_table.T, sie, sde, sir, sdr, wse, wsr)
    return _tc_score(rows)


# speedup vs baseline: 1.0001x; 1.0001x over previous
import functools

import jax
import jax.numpy as jnp
from jax import lax
from jax.experimental import pallas as pl
from jax.experimental.pallas import tpu as pltpu
from jax.experimental.pallas import tpu_sc as plsc

BATCH = 16384
DIM = 64
MARGIN = 1.0
PAIR = 128

NW = 32
NROWS = 1000000

CHUNK = 512
NCH = NROWS // CHUNK
TAILE = NROWS - NCH * CHUNK
CPW = 61
TSLOTS = 62

N_ENT = 4 * BATCH
N_REL = 2 * BATCH
N_ALL = N_ENT + N_REL
DUMP = N_ALL
OUT_ROWS = 100352

LBLK = 512


def _iota16():
    return lax.iota(jnp.int32, 16)


def _lane(vec, j):
    return jnp.sum(jnp.where(_iota16() == j, vec, 0))


def _sc_extract_body(entT, relT, sie, sde, sir, sdr, wse, wsr, out,
                     slab, tslab, lbi, lbd, wsv, staging, dst2d,
                     sem0, sem1, sc0, sc1, sc2, sc3):
    wid = lax.axis_index("s") * 2 + lax.axis_index("c")
    slab_sems = (sem0, sem1)
    sc_sems = (sc0, sc1, sc2, sc3)

    def ws_at(ws_ref, k):
        pltpu.sync_copy(ws_ref.at[pl.ds((k // 16) * 16, 16)], wsv)
        return _lane(wsv[...], k % 16)

    def scatter_wait(r_static, nscat):
        @pl.when(nscat >= 4)
        def _():
            pltpu.make_async_copy(
                staging.at[r_static],
                out.at[dst2d.at[r_static]],
                sc_sems[r_static]).wait()

    def scatter_start(r_static):
        pltpu.make_async_copy(
            staging.at[r_static],
            out.at[dst2d.at[r_static]],
            sc_sems[r_static]).start()

    def window_passes(slab_ref, width, sidx, sdst, s0, s1, e0, live0, state):
        hi = e0 + width

        def cond(st):
            return st[3]

        def body(st):
            wp, blk, nscat, _ = st
            wblk = (16 * wp) // LBLK

            @pl.when(wblk != blk)
            def _():
                pltpu.sync_copy(sidx.at[pl.ds(wblk * LBLK, LBLK)], lbi)
                pltpu.sync_copy(sdst.at[pl.ds(wblk * LBLK, LBLK)], lbd)

            ow = (16 * wp) - wblk * LBLK
            idxw = lbi[pl.ds(ow, 16)]
            dstw = lbd[pl.ds(ow, 16)]
            posv = _iota16() + 16 * wp
            valid = ((posv >= s0) & (posv < s1)
                     & (idxw >= e0) & (idxw < hi))
            cols = jnp.clip(idxw - e0, 0, width - 1)

            r = nscat % 4
            for rr in range(4):
                @pl.when(r == rr)
                def _(rr=rr):
                    scatter_wait(rr, nscat)
            for j in range(16):
                cj = jnp.full((16,), _lane(cols, j), jnp.int32)
                for g in range(4):
                    v = plsc.load_gather(
                        slab_ref, [_iota16() + 16 * g, cj])
                    staging[r, j, pl.ds(16 * g, 16)] = v
            dst2d[r, pl.ds(0, 16)] = jnp.where(valid, dstw, DUMP)
            for rr in range(4):
                @pl.when(r == rr)
                def _(rr=rr):
                    scatter_start(rr)

            bad = jnp.max(jnp.where((idxw >= hi) & (posv < s1), 1, 0))
            done = bad == 0
            wp2 = jnp.where(done, wp + 1, wp)
            more = done & (16 * wp2 < s1)
            return (wp2, wblk, nscat + 1, more)

        wp, blk, nscat, _ = lax.while_loop(
            cond, body, (state[0], state[1], state[2],
                         live0 & (16 * state[0] < s1)))
        return (wp, blk, nscat)

    def phase(tab, sidx, sdst, ws_ref, nscat_in):
        s0 = ws_at(ws_ref, wid)
        s1 = ws_at(ws_ref, wid + 1)
        c_lo = wid * CPW
        c_hi = jnp.minimum(c_lo + CPW + 1, NCH)

        def slab_descs(c, b):
            return [pltpu.make_async_copy(
                tab.at[pl.ds(8 * g, 8), pl.ds(c * CHUNK, CHUNK)],
                slab.at[b, pl.ds(8 * g, 8)], slab_sems[b])
                for g in range(DIM // 8)]

        def slab_start(c, b):
            for d_ in slab_descs(c, b):
                d_.start()

        def slab_wait(c, b):
            for d_ in slab_descs(c, b):
                d_.wait()

        slab_start(c_lo, 0)

        state0 = (s0 // 16, jnp.int32(-1), nscat_in)

        @pl.loop(0, TSLOTS // 2, init_carry=state0)
        def chunk_loop(tt, carry):
            for b in range(2):
                t = 2 * tt + b
                c = c_lo + t

                @pl.when(c + 1 < c_hi)
                def _():
                    slab_start(c + 1, 1 - b)

                live = c < c_hi

                @pl.when(live)
                def _():
                    slab_wait(c, b)

                carry = window_passes(
                    slab.at[b], CHUNK, sidx, sdst, s0, s1,
                    c * CHUNK, live, carry)
            return carry

        wp, blk, nscat = chunk_loop
        live_t = wid == NW - 1

        @pl.when(live_t)
        def _():
            pltpu.sync_copy(tab.at[:, pl.ds(NCH * CHUNK, TAILE)], tslab)

        wp, blk, nscat = window_passes(
            tslab, TAILE, sidx, sdst, s0, s1,
            NCH * CHUNK, live_t, (wp, blk, nscat))
        return nscat

    nscat = phase(entT, sie, sde, wse, jnp.int32(0))
    nscat = phase(relT, sir, sdr, wsr, nscat)

    for d in range(1, 5):
        k = nscat - d
        r = k % 4
        for rr in range(4):
            @pl.when((k >= 0) & (r == rr))
            def _(rr=rr):
                pltpu.make_async_copy(
                    staging.at[rr], out.at[dst2d.at[rr]],
                    sc_sems[rr]).wait()


def _make_sc_extract():
    mesh = plsc.VectorSubcoreMesh(core_axis_name="c", subcore_axis_name="s")
    return functools.partial(
        pl.kernel, mesh=mesh,
        compiler_params=pltpu.CompilerParams(needs_layout_passes=False),
        out_type=jax.ShapeDtypeStruct((OUT_ROWS, PAIR), jnp.float32),
        scratch_types=[
            pltpu.VMEM((2, DIM, CHUNK), jnp.float32),
            pltpu.VMEM((DIM, TAILE), jnp.float32),
            pltpu.VMEM((LBLK,), jnp.int32),
            pltpu.VMEM((LBLK,), jnp.int32),
            pltpu.VMEM((16,), jnp.int32),
            pltpu.VMEM((4, 16, PAIR), jnp.float32),
            pltpu.VMEM((4, 16), jnp.int32),
            pltpu.SemaphoreType.DMA,
            pltpu.SemaphoreType.DMA,
            pltpu.SemaphoreType.DMA,
            pltpu.SemaphoreType.DMA,
            pltpu.SemaphoreType.DMA,
            pltpu.SemaphoreType.DMA,
        ],
    )(_sc_extract_body)


_sc_extract = _make_sc_extract()

CB = 2048
NBLK = BATCH // CB


def _score_body(ph, pt, pr, nh, nt, nr, out):
    k = pl.program_id(0)

    def score(h_ref, t_ref, r_ref):
        h = h_ref[...][:, :DIM]
        t = t_ref[...][:, :DIM]
        r = r_ref[...][:, :DIM]
        hn = h / jnp.sqrt(jnp.sum(h * h, axis=1, keepdims=True))
        tn = t / jnp.sqrt(jnp.sum(t * t, axis=1, keepdims=True))
        diff = hn + r - tn
        return jnp.sqrt(jnp.sum(diff * diff, axis=1))

    ps = score(ph, pt, pr)
    ns = score(nh, nt, nr)
    part = jnp.sum(jnp.maximum(MARGIN + ps - ns, 0.0)).reshape(1, 1)
    prev = jnp.where(k == 0, jnp.zeros((1, 1), jnp.float32), out[...])
    total = prev + part
    out[...] = jnp.where(k == NBLK - 1, total / BATCH, total)


def _tc_score(rows):
    def blk(off):
        return pl.BlockSpec((CB, PAIR), lambda k, o=off: (k + o, 0))

    out = pl.pallas_call(
        _score_body,
        grid=(NBLK,),
        in_specs=[blk(0), blk(NBLK), blk(4 * NBLK), blk(2 * NBLK),
                  blk(3 * NBLK), blk(5 * NBLK)],
        out_specs=pl.BlockSpec((1, 1), lambda k: (0, 0)),
        out_shape=jax.ShapeDtypeStruct((1, 1), jnp.float32),
    )(rows, rows, rows, rows, rows, rows)
    return out.reshape(())


def kernel(pos_x, neg_x, ent_table, rel_table):
    ent_idx = jnp.concatenate(
        [pos_x[:, 0], pos_x[:, 1], neg_x[:, 0], neg_x[:, 1]])
    rel_idx = jnp.concatenate([pos_x[:, 2], neg_x[:, 2]])

    eperm = jnp.argsort(ent_idx)
    sie = ent_idx[eperm]
    sde = eperm.astype(jnp.int32)
    rperm = jnp.argsort(rel_idx)
    sir = rel_idx[rperm]
    sdr = rperm.astype(jnp.int32) + N_ENT

    bounds = jnp.concatenate(
        [jnp.arange(32, dtype=jnp.int32) * (CPW * CHUNK),
         jnp.array([NROWS], jnp.int32)])
    wse = jnp.zeros((48,), jnp.int32).at[:33].set(
        jnp.searchsorted(sie, bounds).astype(jnp.int32))
    wsr = jnp.zeros((48,), jnp.int32).at[:33].set(
        jnp.searchsorted(sir, bounds).astype(jnp.int32))

    rows = _sc_extract(ent_table.T, rel_table.T, sie, sde, sir, sdr, wse, wsr)
    return _tc_score(rows)
